# baseline (speedup 1.0000x reference)
import functools

import jax
import jax.numpy as jnp
from jax import lax
from jax.experimental import pallas as pl
from jax.experimental.pallas import tpu as pltpu
from jax.experimental.pallas import tpu_sc as plsc

NENTITY = 1000000
NRELATION = 1000
D = 64
B = 16384
L = 16
NC, NS = 2, 16
NW = NC * NS
BPW = B // NW
NIR = BPW // 128
NCHUNK = BPW // L
LOOKAHEAD = 8


def _sc_score(heads2, rels2, tails2, ent_t, rel_t, gamma_arr):
    mesh = plsc.VectorSubcoreMesh(
        core_axis_name="c", subcore_axis_name="s", num_cores=NC, num_subcores=NS
    )

    @functools.partial(
        pl.kernel,
        out_type=jax.ShapeDtypeStruct((B,), jnp.float32),
        mesh=mesh,
        compiler_params=pltpu.CompilerParams(
            needs_layout_passes=False, use_tc_tiling_on_sc=False
        ),
        scratch_types=dict(
            h_ix=pltpu.VMEM((NIR, 128), jnp.int32),
            t_ix=pltpu.VMEM((NIR, 128), jnp.int32),
            r_ix=pltpu.VMEM((NIR, 128), jnp.int32),
            h_all=pltpu.VMEM((D, BPW), jnp.float32),
            t_all=pltpu.VMEM((D, BPW), jnp.float32),
            r_all=pltpu.VMEM((D, BPW), jnp.float32),
            out_v=pltpu.VMEM((BPW,), jnp.float32),
            gamma_v=pltpu.VMEM((L,), jnp.float32),
            sem0=pltpu.SemaphoreType.DMA,
            sem1=pltpu.SemaphoreType.DMA,
        ),
    )
    def body(heads_hbm, rels_hbm, tails_hbm, ent_hbm, rel_hbm, gamma_hbm,
             out_hbm, h_ix, t_ix, r_ix, h_all, t_all, r_all, out_v,
             gamma_v, sem0, sem1):
        wid = lax.axis_index("s") * NC + lax.axis_index("c")
        rbase = wid * NIR
        sems = (sem0, sem1)

        pltpu.sync_copy(heads_hbm.at[pl.ds(rbase, NIR)], h_ix)
        pltpu.sync_copy(tails_hbm.at[pl.ds(rbase, NIR)], t_ix)
        pltpu.sync_copy(rels_hbm.at[pl.ds(rbase, NIR)], r_ix)
        pltpu.sync_copy(gamma_hbm, gamma_v)

        def issue(d):
            sem = sem0
            for r in range(NIR):
                dsl = pl.ds(r * 128, 128)
                pltpu.async_copy(ent_hbm.at[d].at[h_ix.at[r]],
                                 h_all.at[d].at[dsl], sem)
                pltpu.async_copy(ent_hbm.at[d].at[t_ix.at[r]],
                                 t_all.at[d].at[dsl], sem)
                pltpu.async_copy(rel_hbm.at[d].at[r_ix.at[r]],
                                 r_all.at[d].at[dsl], sem)

        def drain(d):
            sem = sem0
            for r in range(NIR):
                dsl = pl.ds(r * 128, 128)
                pltpu.make_async_copy(ent_hbm.at[d].at[h_ix.at[r]],
                                      h_all.at[d].at[dsl], sem).wait()
                pltpu.make_async_copy(ent_hbm.at[d].at[t_ix.at[r]],
                                      t_all.at[d].at[dsl], sem).wait()
                pltpu.make_async_copy(rel_hbm.at[d].at[r_ix.at[r]],
                                      r_all.at[d].at[dsl], sem).wait()

        def issue_body(d, carry):
            issue(d)

            @pl.when(d >= LOOKAHEAD)
            def _():
                drain(d - LOOKAHEAD)

            return carry

        lax.fori_loop(0, D, issue_body, 0)
        lax.fori_loop(D - LOOKAHEAD, D, lambda d, cy: (drain(d), cy)[1], 0)

        gvec = gamma_v[...]

        def chunk_body(c, carry):
            dsl = pl.ds(c * L, L)
            acc = None
            for d in range(D):
                term = jnp.abs(h_all[d, dsl] + r_all[d, dsl] - t_all[d, dsl])
                acc = term if acc is None else acc + term
            out_v[dsl] = gvec - acc
            return carry

        lax.fori_loop(0, NCHUNK, chunk_body, 0)

        pltpu.sync_copy(out_v, out_hbm.at[pl.ds(wid * BPW, BPW)])

    return body(heads2, rels2, tails2, ent_t, rel_t, gamma_arr)


def kernel(sample, entity_embedding, relation_embedding, gamma):
    heads2 = sample[:, 0].reshape(B // 128, 128)
    rels2 = sample[:, 1].reshape(B // 128, 128)
    tails2 = sample[:, 2].reshape(B // 128, 128)
    ent_t = entity_embedding.T
    rel_t = relation_embedding.T
    gamma_arr = jnp.full((L,), gamma, dtype=jnp.float32)
    score = _sc_score(heads2, rels2, tails2, ent_t, rel_t, gamma_arr)
    return score.reshape(B, 1)


# speedup vs baseline: 3.2957x; 3.2957x over previous
import functools

import jax
import jax.numpy as jnp
from jax import lax
from jax.experimental import pallas as pl
from jax.experimental.pallas import tpu as pltpu
from jax.experimental.pallas import tpu_sc as plsc

NENTITY = 1000000
NRELATION = 1000
D = 64
B = 16384
L = 16
NC, NS = 2, 16
NW = NC * NS
NPAIR = NENTITY // 2
NCOL = NENTITY // 128
TAILC = NENTITY - NCOL * 128
COLS_PW = (NCOL + NW) // NW

BPW = B // NW
C = 16
NCHUNK = BPW // C
NBUF = 2


def _repack(ent_t, tailp):
    mesh = plsc.VectorSubcoreMesh(
        core_axis_name="c", subcore_axis_name="s", num_cores=NC, num_subcores=NS
    )

    @functools.partial(
        pl.kernel,
        out_type=jax.ShapeDtypeStruct((NPAIR, 2 * D), jnp.float32),
        mesh=mesh,
        compiler_params=pltpu.CompilerParams(
            needs_layout_passes=False, use_tc_tiling_on_sc=True
        ),
        scratch_types=dict(
            blk=pltpu.VMEM((NBUF, D, 128), jnp.float32),
            stg=pltpu.VMEM((NBUF, D, 2 * D), jnp.float32),
            isem=pltpu.SemaphoreType.DMA,
            osem=pltpu.SemaphoreType.DMA,
        ),
    )
    def body(ent_hbm, tailp_hbm, out_hbm, blk, stg, isem, osem):
        wid = lax.axis_index("s") * NC + lax.axis_index("c")
        c0 = wid * COLS_PW
        lane = lax.iota(jnp.int32, L)

        def col_off(c):
            return pl.multiple_of(c * 128, 128)

        def row_off(c):
            return pl.multiple_of(c * D, 8)

        def issue_in(c, buf):
            pltpu.async_copy(ent_hbm.at[:, pl.ds(col_off(c), 128)],
                             blk.at[buf], isem)

        def transpose(buf):
            bb = blk.at[buf]
            sb = stg.at[buf]
            for p in range(D):
                for k in range(8):
                    e = 2 * p + k // 4
                    dv = (k % 4) * L + lane
                    sb[p, pl.ds(k * L, L)] = plsc.load_gather(
                        bb, [dv, jnp.full((L,), e, jnp.int32)])

        def flush_out(c, buf):
            pltpu.async_copy(stg.at[buf], out_hbm.at[pl.ds(row_off(c), D)],
                             osem)

        def wait_in(c, buf):
            pltpu.make_async_copy(ent_hbm.at[:, pl.ds(0, 128)],
                                  blk.at[buf], isem).wait()

        def wait_out(c, buf):
            pltpu.make_async_copy(stg.at[buf], out_hbm.at[pl.ds(0, D)],
                                  osem).wait()

        nc = jnp.minimum(c0 + COLS_PW, NCOL) - c0

        @pl.when(nc > 0)
        def _():
            issue_in(c0, 0)

            def loop(i, carry):
                c = c0 + i
                b = lax.rem(i, 2)

                @pl.when(i + 1 < nc)
                def _():
                    issue_in(c + 1, 1 - b)

                wait_in(c, b)

                @pl.when(i >= 2)
                def _():
                    wait_out(c - 2, b)

                transpose(b)
                flush_out(c, b)
                return carry

            lax.fori_loop(0, nc, loop, 0)

            @pl.when(nc >= 2)
            def _():
                wait_out(0, 0)

            @pl.when(nc >= 1)
            def _():
                wait_out(0, 1)

        @pl.when(wid == NW - 1)
        def _():
            pltpu.sync_copy(tailp_hbm, stg.at[0].at[pl.ds(0, TAILC // 2)])
            pltpu.sync_copy(stg.at[0].at[pl.ds(0, TAILC // 2)],
                            out_hbm.at[pl.ds(NCOL * D, TAILC // 2)])

    return body(ent_t, tailp)


def _score(heads2, rels2, tails2, pairs, relp, gamma_arr):
    mesh = plsc.VectorSubcoreMesh(
        core_axis_name="c", subcore_axis_name="s", num_cores=NC, num_subcores=NS
    )

    NCPC = 128
    NGC = NCPC // L
    NCH = BPW // NCPC

    @functools.partial(
        pl.kernel,
        out_type=jax.ShapeDtypeStruct((B,), jnp.float32),
        mesh=mesh,
        compiler_params=pltpu.CompilerParams(
            needs_layout_passes=False, use_tc_tiling_on_sc=True
        ),
        scratch_types=dict(
            h_idv=pltpu.VMEM((BPW,), jnp.int32),
            t_idv=pltpu.VMEM((BPW,), jnp.int32),
            r_idv=pltpu.VMEM((BPW,), jnp.int32),
            h_ix=pltpu.VMEM((NBUF, NCPC), jnp.int32),
            t_ix=pltpu.VMEM((NBUF, NCPC), jnp.int32),
            h_rows=pltpu.VMEM((NBUF, NCPC, 2 * D), jnp.float32),
            t_rows=pltpu.VMEM((NBUF, NCPC, 2 * D), jnp.float32),
            r_rows=pltpu.VMEM((NBUF, NCPC, 2 * D), jnp.float32),
            out_v=pltpu.VMEM((BPW,), jnp.float32),
            gamma_v=pltpu.VMEM((L,), jnp.float32),
            sem0=pltpu.SemaphoreType.DMA,
            sem1=pltpu.SemaphoreType.DMA,
        ),
    )
    def body(heads_hbm, rels_hbm, tails_hbm, pair_hbm, rel_hbm, gamma_hbm,
             out_hbm, h_idv, t_idv, r_idv, h_ix, t_ix, h_rows, t_rows,
             r_rows, out_v, gamma_v, sem0, sem1):
        wid = lax.axis_index("s") * NC + lax.axis_index("c")
        base = wid * BPW
        sems = (sem0, sem1)
        lane = lax.iota(jnp.int32, L)
        one = jnp.full((L,), 1, jnp.int32)

        pltpu.sync_copy(heads_hbm.at[pl.ds(base, BPW)], h_idv)
        pltpu.sync_copy(tails_hbm.at[pl.ds(base, BPW)], t_idv)
        pltpu.sync_copy(rels_hbm.at[pl.ds(base, BPW)], r_idv)
        pltpu.sync_copy(gamma_hbm, gamma_v)

        def issue(c, buf):
            sem = sems[buf]
            for g in range(NGC):
                dsl = pl.ds(g * L, L)
                h_ix.at[buf][dsl] = h_idv[pl.ds(c * NCPC + g * L, L)] >> 1
                t_ix.at[buf][dsl] = t_idv[pl.ds(c * NCPC + g * L, L)] >> 1
            pltpu.async_copy(pair_hbm.at[h_ix.at[buf]], h_rows.at[buf], sem)
            pltpu.async_copy(pair_hbm.at[t_ix.at[buf]], t_rows.at[buf], sem)
            pltpu.async_copy(rel_hbm.at[r_idv.at[pl.ds(c * NCPC, NCPC)]],
                             r_rows.at[buf], sem)

        def drain(c, buf):
            sem = sems[buf]
            pltpu.make_async_copy(pair_hbm.at[h_ix.at[buf]],
                                  h_rows.at[buf], sem).wait()
            pltpu.make_async_copy(pair_hbm.at[t_ix.at[buf]],
                                  t_rows.at[buf], sem).wait()
            pltpu.make_async_copy(rel_hbm.at[r_idv.at[pl.ds(0, NCPC)]],
                                  r_rows.at[buf], sem).wait()

        issue(0, 0)
        issue(1, 1)

        gvec = gamma_v[...]

        def half_body(half, carry):
            for b in range(NBUF):
                c = half * NBUF + b
                drain(c, b)

                def group_body(g, cy):
                    jl = g * L + lane
                    tsl = pl.ds(c * NCPC + g * L, L)
                    hoff = (h_idv[tsl] & one) * D
                    toff = (t_idv[tsl] & one) * D
                    acc = None
                    for d in range(D):
                        hv = plsc.load_gather(h_rows.at[b], [jl, hoff + d])
                        tv = plsc.load_gather(t_rows.at[b], [jl, toff + d])
                        rv = plsc.load_gather(r_rows.at[b],
                                              [jl, jnp.full((L,), d, jnp.int32)])
                        term = jnp.abs(hv + rv - tv)
                        acc = term if acc is None else acc + term
                    out_v[tsl] = gvec - acc
                    return cy

                lax.fori_loop(0, NGC, group_body, 0)

                @pl.when(c + NBUF < NCH)
                def _():
                    issue(c + NBUF, b)

            return carry

        lax.fori_loop(0, NCH // NBUF, half_body, 0)

        pltpu.sync_copy(out_v, out_hbm.at[pl.ds(base, BPW)])

    return body(heads2, rels2, tails2, pairs, relp, gamma_arr)


def kernel(sample, entity_embedding, relation_embedding, gamma):
    heads2 = sample[:, 0]
    rels2 = sample[:, 1]
    tails2 = sample[:, 2]
    ent_t = entity_embedding.T
    tailp = entity_embedding[NCOL * 128:].reshape(TAILC // 2, 2 * D)
    pairs = _repack(ent_t, tailp)
    relp = jnp.pad(relation_embedding, ((0, 0), (0, D)))
    gamma_arr = jnp.full((L,), gamma, dtype=jnp.float32)
    score = _score(heads2, rels2, tails2, pairs, relp, gamma_arr)
    return score.reshape(B, 1)


# speedup vs baseline: 4.0660x; 1.2337x over previous
import functools

import jax
import jax.numpy as jnp
from jax import lax
from jax.experimental import pallas as pl
from jax.experimental.pallas import tpu as pltpu
from jax.experimental.pallas import tpu_sc as plsc

NENTITY = 1000000
NRELATION = 1000
D = 64
B = 16384
L = 16
NC, NS = 2, 16
NW = NC * NS
NPAIR = NENTITY // 2
NCOL = NENTITY // 128
TAILC = NENTITY - NCOL * 128
COLS_PW = (NCOL + NW) // NW

BPW = B // NW
C = 16
NCHUNK = BPW // C
NBUF = 2


def _repack(ent_t, tailp):
    mesh = plsc.VectorSubcoreMesh(
        core_axis_name="c", subcore_axis_name="s", num_cores=NC, num_subcores=NS
    )

    @functools.partial(
        pl.kernel,
        out_type=jax.ShapeDtypeStruct((NPAIR, 2 * D), jnp.float32),
        mesh=mesh,
        compiler_params=pltpu.CompilerParams(
            needs_layout_passes=False, use_tc_tiling_on_sc=True
        ),
        scratch_types=dict(
            blk=pltpu.VMEM((NBUF, D, 128), jnp.float32),
            stg=pltpu.VMEM((NBUF, D, 2 * D), jnp.float32),
            isem=pltpu.SemaphoreType.DMA,
            osem=pltpu.SemaphoreType.DMA,
        ),
    )
    def body(ent_hbm, tailp_hbm, out_hbm, blk, stg, isem, osem):
        wid = lax.axis_index("s") * NC + lax.axis_index("c")
        c0 = wid * COLS_PW
        lane = lax.iota(jnp.int32, L)

        def col_off(c):
            return pl.multiple_of(c * 128, 128)

        def row_off(c):
            return pl.multiple_of(c * D, 8)

        def issue_in(c, buf):
            pltpu.async_copy(ent_hbm.at[:, pl.ds(col_off(c), 128)],
                             blk.at[buf], isem)

        ebr = lax.shift_right_logical(lane, 1)
        ebc = (lane & jnp.full((L,), 1, jnp.int32)) * D

        def transpose(buf):
            bb = blk.at[buf]
            sb = stg.at[buf]
            for d in range(D):
                for q in range(8):
                    row = bb[d, pl.ds(q * L, L)]
                    plsc.store_scatter(sb, [ebr + q * 8, ebc + d], row)

        def flush_out(c, buf):
            pltpu.async_copy(stg.at[buf], out_hbm.at[pl.ds(row_off(c), D)],
                             osem)

        def wait_in(c, buf):
            pltpu.make_async_copy(ent_hbm.at[:, pl.ds(0, 128)],
                                  blk.at[buf], isem).wait()

        def wait_out(c, buf):
            pltpu.make_async_copy(stg.at[buf], out_hbm.at[pl.ds(0, D)],
                                  osem).wait()

        nc = jnp.minimum(c0 + COLS_PW, NCOL) - c0

        @pl.when(nc > 0)
        def _():
            issue_in(c0, 0)

            def loop(i, carry):
                c = c0 + i
                b = lax.rem(i, 2)

                @pl.when(i + 1 < nc)
                def _():
                    issue_in(c + 1, 1 - b)

                wait_in(c, b)

                @pl.when(i >= 2)
                def _():
                    wait_out(c - 2, b)

                transpose(b)
                flush_out(c, b)
                return carry

            lax.fori_loop(0, nc, loop, 0)

            @pl.when(nc >= 2)
            def _():
                wait_out(0, 0)

            @pl.when(nc >= 1)
            def _():
                wait_out(0, 1)

        @pl.when(wid == NW - 1)
        def _():
            pltpu.sync_copy(tailp_hbm, stg.at[0].at[pl.ds(0, TAILC // 2)])
            pltpu.sync_copy(stg.at[0].at[pl.ds(0, TAILC // 2)],
                            out_hbm.at[pl.ds(NCOL * D, TAILC // 2)])

    return body(ent_t, tailp)


def _score(heads2, rels2, tails2, pairs, relp, gamma_arr):
    mesh = plsc.VectorSubcoreMesh(
        core_axis_name="c", subcore_axis_name="s", num_cores=NC, num_subcores=NS
    )

    NCPC = 128
    NGC = NCPC // L
    NCH = BPW // NCPC

    @functools.partial(
        pl.kernel,
        out_type=jax.ShapeDtypeStruct((B,), jnp.float32),
        mesh=mesh,
        compiler_params=pltpu.CompilerParams(
            needs_layout_passes=False, use_tc_tiling_on_sc=True
        ),
        scratch_types=dict(
            h_idv=pltpu.VMEM((BPW,), jnp.int32),
            t_idv=pltpu.VMEM((BPW,), jnp.int32),
            r_idv=pltpu.VMEM((BPW,), jnp.int32),
            h_ix=pltpu.VMEM((NBUF, NCPC), jnp.int32),
            t_ix=pltpu.VMEM((NBUF, NCPC), jnp.int32),
            h_rows=pltpu.VMEM((NBUF, NCPC, 2 * D), jnp.float32),
            t_rows=pltpu.VMEM((NBUF, NCPC, 2 * D), jnp.float32),
            r_rows=pltpu.VMEM((NBUF, NCPC, 2 * D), jnp.float32),
            out_v=pltpu.VMEM((BPW,), jnp.float32),
            gamma_v=pltpu.VMEM((L,), jnp.float32),
            sem0=pltpu.SemaphoreType.DMA,
            sem1=pltpu.SemaphoreType.DMA,
        ),
    )
    def body(heads_hbm, rels_hbm, tails_hbm, pair_hbm, rel_hbm, gamma_hbm,
             out_hbm, h_idv, t_idv, r_idv, h_ix, t_ix, h_rows, t_rows,
             r_rows, out_v, gamma_v, sem0, sem1):
        wid = lax.axis_index("s") * NC + lax.axis_index("c")
        base = wid * BPW
        sems = (sem0, sem1)
        lane = lax.iota(jnp.int32, L)
        one = jnp.full((L,), 1, jnp.int32)

        pltpu.sync_copy(heads_hbm.at[pl.ds(base, BPW)], h_idv)
        pltpu.sync_copy(tails_hbm.at[pl.ds(base, BPW)], t_idv)
        pltpu.sync_copy(rels_hbm.at[pl.ds(base, BPW)], r_idv)
        pltpu.sync_copy(gamma_hbm, gamma_v)

        def issue(c, buf):
            sem = sems[buf]
            for g in range(NGC):
                dsl = pl.ds(g * L, L)
                h_ix.at[buf][dsl] = h_idv[pl.ds(c * NCPC + g * L, L)] >> 1
                t_ix.at[buf][dsl] = t_idv[pl.ds(c * NCPC + g * L, L)] >> 1
            pltpu.async_copy(pair_hbm.at[h_ix.at[buf]], h_rows.at[buf], sem)
            pltpu.async_copy(pair_hbm.at[t_ix.at[buf]], t_rows.at[buf], sem)
            pltpu.async_copy(rel_hbm.at[r_idv.at[pl.ds(c * NCPC, NCPC)]],
                             r_rows.at[buf], sem)

        def drain(c, buf):
            sem = sems[buf]
            pltpu.make_async_copy(pair_hbm.at[h_ix.at[buf]],
                                  h_rows.at[buf], sem).wait()
            pltpu.make_async_copy(pair_hbm.at[t_ix.at[buf]],
                                  t_rows.at[buf], sem).wait()
            pltpu.make_async_copy(rel_hbm.at[r_idv.at[pl.ds(0, NCPC)]],
                                  r_rows.at[buf], sem).wait()

        issue(0, 0)
        issue(1, 1)

        gvec = gamma_v[...]

        def half_body(half, carry):
            for b in range(NBUF):
                c = half * NBUF + b
                drain(c, b)

                def group_body(g, cy):
                    jl = g * L + lane
                    tsl = pl.ds(c * NCPC + g * L, L)
                    hoff = (h_idv[tsl] & one) * D
                    toff = (t_idv[tsl] & one) * D
                    acc = None
                    for d in range(D):
                        hv = plsc.load_gather(h_rows.at[b], [jl, hoff + d])
                        tv = plsc.load_gather(t_rows.at[b], [jl, toff + d])
                        rv = plsc.load_gather(r_rows.at[b],
                                              [jl, jnp.full((L,), d, jnp.int32)])
                        term = jnp.abs(hv + rv - tv)
                        acc = term if acc is None else acc + term
                    out_v[tsl] = gvec - acc
                    return cy

                lax.fori_loop(0, NGC, group_body, 0)

                @pl.when(c + NBUF < NCH)
                def _():
                    issue(c + NBUF, b)

            return carry

        lax.fori_loop(0, NCH // NBUF, half_body, 0)

        pltpu.sync_copy(out_v, out_hbm.at[pl.ds(base, BPW)])

    return body(heads2, rels2, tails2, pairs, relp, gamma_arr)


def kernel(sample, entity_embedding, relation_embedding, gamma):
    heads2 = sample[:, 0]
    rels2 = sample[:, 1]
    tails2 = sample[:, 2]
    ent_t = entity_embedding.T
    tailp = entity_embedding[NCOL * 128:].reshape(TAILC // 2, 2 * D)
    pairs = _repack(ent_t, tailp)
    relp = jnp.pad(relation_embedding, ((0, 0), (0, D)))
    gamma_arr = jnp.full((L,), gamma, dtype=jnp.float32)
    score = _score(heads2, rels2, tails2, pairs, relp, gamma_arr)
    return score.reshape(B, 1)


# speedup vs baseline: 7.7630x; 1.9093x over previous
import functools

import jax
import jax.numpy as jnp
from jax import lax
from jax.experimental import pallas as pl
from jax.experimental.pallas import tpu as pltpu
from jax.experimental.pallas import tpu_sc as plsc

NENTITY = 1000000
NRELATION = 1000
D = 64
B = 16384
L = 16
NC, NS = 2, 16
NW = NC * NS
BPW = B // NW
C = 128
NCH = BPW // C
NG = C // L
NBUF = 2


def _sc_score(heads, rels, tails, ent, relp, gamma_arr):
    mesh = plsc.VectorSubcoreMesh(
        core_axis_name="c", subcore_axis_name="s", num_cores=NC, num_subcores=NS
    )

    @functools.partial(
        pl.kernel,
        out_type=jax.ShapeDtypeStruct((B,), jnp.float32),
        mesh=mesh,
        compiler_params=pltpu.CompilerParams(
            needs_layout_passes=False, use_tc_tiling_on_sc=False
        ),
        scratch_types=dict(
            h_idv=pltpu.VMEM((BPW,), jnp.int32),
            t_idv=pltpu.VMEM((BPW,), jnp.int32),
            r_idv=pltpu.VMEM((BPW,), jnp.int32),
            h_ix=pltpu.VMEM((NBUF, C), jnp.int32),
            t_ix=pltpu.VMEM((NBUF, C), jnp.int32),
            r_ix=pltpu.VMEM((NBUF, C), jnp.int32),
            h_rows=pltpu.VMEM((NBUF, C, D), jnp.float32),
            t_rows=pltpu.VMEM((NBUF, C, D), jnp.float32),
            r_rows=pltpu.VMEM((NBUF, C, 2 * D), jnp.float32),
            out_v=pltpu.VMEM((BPW,), jnp.float32),
            gamma_v=pltpu.VMEM((L,), jnp.float32),
            sem0=pltpu.SemaphoreType.DMA,
            sem1=pltpu.SemaphoreType.DMA,
        ),
    )
    def body(heads_hbm, rels_hbm, tails_hbm, ent_hbm, rel_hbm, gamma_hbm,
             out_hbm, h_idv, t_idv, r_idv, h_ix, t_ix, r_ix, h_rows,
             t_rows, r_rows, out_v, gamma_v, sem0, sem1):
        wid = lax.axis_index("s") * NC + lax.axis_index("c")
        base = wid * BPW
        sems = (sem0, sem1)
        lane = lax.iota(jnp.int32, L)

        pltpu.sync_copy(heads_hbm.at[pl.ds(base, BPW)], h_idv)
        pltpu.sync_copy(tails_hbm.at[pl.ds(base, BPW)], t_idv)
        pltpu.sync_copy(rels_hbm.at[pl.ds(base, BPW)], r_idv)
        pltpu.sync_copy(gamma_hbm, gamma_v)

        def issue(c, buf):
            sem = sems[buf]
            for g in range(NG):
                src = pl.ds(c * C + g * L, L)
                dst = pl.ds(g * L, L)
                h_ix.at[buf][dst] = h_idv[src]
                t_ix.at[buf][dst] = t_idv[src]
                r_ix.at[buf][dst] = r_idv[src]
            pltpu.async_copy(ent_hbm.at[h_ix.at[buf]], h_rows.at[buf], sem)
            pltpu.async_copy(ent_hbm.at[t_ix.at[buf]], t_rows.at[buf], sem)
            pltpu.async_copy(rel_hbm.at[r_ix.at[buf]], r_rows.at[buf], sem)

        def drain(buf):
            sem = sems[buf]
            pltpu.make_async_copy(ent_hbm.at[h_ix.at[buf]],
                                  h_rows.at[buf], sem).wait()
            pltpu.make_async_copy(ent_hbm.at[t_ix.at[buf]],
                                  t_rows.at[buf], sem).wait()
            pltpu.make_async_copy(rel_hbm.at[r_ix.at[buf]],
                                  r_rows.at[buf], sem).wait()

        issue(0, 0)
        issue(1, 1)

        gvec = gamma_v[...]

        def half_body(half, carry):
            for b in range(NBUF):
                c = half * NBUF + b
                drain(b)

                def group_body(g, cy):
                    jl = g * L + lane
                    acc = None
                    for d in range(D):
                        dv = jnp.full((L,), d, jnp.int32)
                        hv = plsc.load_gather(h_rows.at[b], [jl, dv])
                        tv = plsc.load_gather(t_rows.at[b], [jl, dv])
                        rv = plsc.load_gather(r_rows.at[b], [jl, dv])
                        term = jnp.abs(hv + rv - tv)
                        acc = term if acc is None else acc + term
                    out_v[pl.ds(c * C + g * L, L)] = gvec - acc
                    return cy

                lax.fori_loop(0, NG, group_body, 0)

                @pl.when(c + NBUF < NCH)
                def _():
                    issue(c + NBUF, b)

            return carry

        lax.fori_loop(0, NCH // NBUF, half_body, 0)

        pltpu.sync_copy(out_v, out_hbm.at[pl.ds(base, BPW)])

    return body(heads, rels, tails, ent, relp, gamma_arr)


def kernel(sample, entity_embedding, relation_embedding, gamma):
    heads = sample[:, 0]
    rels = sample[:, 1]
    tails = sample[:, 2]
    relp = jnp.pad(relation_embedding, ((0, 0), (0, D)))
    gamma_arr = jnp.full((L,), gamma, dtype=jnp.float32)
    score = _sc_score(heads, rels, tails, entity_embedding, relp, gamma_arr)
    return score.reshape(B, 1)


# speedup vs baseline: 12.4886x; 1.6087x over previous
import functools

import jax
import jax.numpy as jnp
from jax import lax
from jax.experimental import pallas as pl
from jax.experimental.pallas import tpu as pltpu
from jax.experimental.pallas import tpu_sc as plsc

NENTITY = 1000000
NRELATION = 1000
D = 64
B = 16384
L = 16
NC, NS = 2, 16
NW = NC * NS
BPW = B // NW
C = 16
NCHUNK = BPW // C
NBUF = 2


def _sc_score(heads, rels, tails, ent, relp, gamma_arr):
    mesh = plsc.VectorSubcoreMesh(
        core_axis_name="c", subcore_axis_name="s", num_cores=NC, num_subcores=NS
    )

    @functools.partial(
        pl.kernel,
        out_type=jax.ShapeDtypeStruct((B,), jnp.float32),
        mesh=mesh,
        compiler_params=pltpu.CompilerParams(needs_layout_passes=False),
        scratch_types=dict(
            r_ids=pltpu.VMEM((BPW,), jnp.int32),
            h_idv=pltpu.VMEM((BPW,), jnp.int32),
            t_idv=pltpu.VMEM((BPW,), jnp.int32),
            h_rows=pltpu.VMEM((NBUF, C, D), jnp.float32),
            t_rows=pltpu.VMEM((NBUF, C, D), jnp.float32),
            r_rows=pltpu.VMEM((NBUF, C, 2 * D), jnp.float32),
            out_v=pltpu.VMEM((BPW,), jnp.float32),
            gamma_v=pltpu.VMEM((L,), jnp.float32),
            sem0=pltpu.SemaphoreType.DMA,
            sem1=pltpu.SemaphoreType.DMA,
        ),
    )
    def body(heads_hbm, rels_hbm, tails_hbm, ent_hbm, rel_hbm, gamma_hbm,
             out_hbm, r_ids, h_idv, t_idv, h_rows, t_rows,
             r_rows, out_v, gamma_v, sem0, sem1):
        wid = lax.axis_index("s") * NC + lax.axis_index("c")
        base = wid * BPW
        sems = (sem0, sem1)

        pltpu.sync_copy(heads_hbm.at[pl.ds(base, BPW)], h_idv)
        pltpu.sync_copy(tails_hbm.at[pl.ds(base, BPW)], t_idv)
        pltpu.sync_copy(rels_hbm.at[pl.ds(base, BPW)], r_ids)
        pltpu.sync_copy(gamma_hbm, gamma_v)

        def issue(c, buf):
            sem = sems[buf]
            hv16 = h_idv[pl.ds(c * C, C)]
            tv16 = t_idv[pl.ds(c * C, C)]
            for j in range(C):
                h = hv16[j]
                t = tv16[j]
                pltpu.async_copy(ent_hbm.at[pl.ds(h, 1)],
                                 h_rows.at[buf].at[pl.ds(j, 1)], sem)
                pltpu.async_copy(ent_hbm.at[pl.ds(t, 1)],
                                 t_rows.at[buf].at[pl.ds(j, 1)], sem)
            pltpu.async_copy(rel_hbm.at[r_ids.at[pl.ds(c * C, C)]],
                             r_rows.at[buf], sem)

        def drain(c, buf):
            sem = sems[buf]
            for j in range(C):
                pltpu.make_async_copy(ent_hbm.at[pl.ds(0, 1)],
                                      h_rows.at[buf].at[pl.ds(j, 1)], sem).wait()
                pltpu.make_async_copy(ent_hbm.at[pl.ds(0, 1)],
                                      t_rows.at[buf].at[pl.ds(j, 1)], sem).wait()
            pltpu.make_async_copy(rel_hbm.at[r_ids.at[pl.ds(c * C, C)]],
                                  r_rows.at[buf], sem).wait()

        issue(0, 0)
        issue(1, 1)

        lane = lax.iota(jnp.int32, L)
        gvec = gamma_v[...]

        def chunk_body(half, carry):
            for b in range(NBUF):
                c = half * NBUF + b
                drain(c, b)
                acc = None
                for e in range(D):
                    ev = jnp.full((L,), e, jnp.int32)
                    hv = plsc.load_gather(h_rows.at[b], [lane, ev])
                    tv = plsc.load_gather(t_rows.at[b], [lane, ev])
                    rv = plsc.load_gather(r_rows.at[b], [lane, ev])
                    term = jnp.abs(hv + rv - tv)
                    acc = term if acc is None else acc + term
                out_v[pl.ds(c * C, C)] = gvec - acc

                @pl.when(c + NBUF < NCHUNK)
                def _():
                    issue(c + NBUF, b)

            return carry

        lax.fori_loop(0, NCHUNK // NBUF, chunk_body, 0)

        pltpu.sync_copy(out_v, out_hbm.at[pl.ds(base, BPW)])

    return body(heads, rels, tails, ent, relp, gamma_arr)


def kernel(sample, entity_embedding, relation_embedding, gamma):
    heads = sample[:, 0]
    rels = sample[:, 1]
    tails = sample[:, 2]
    relp = jnp.pad(relation_embedding, ((0, 0), (0, D)))
    gamma_arr = jnp.full((L,), gamma, dtype=jnp.float32)
    score = _sc_score(heads, rels, tails, entity_embedding, relp, gamma_arr)
    return score.reshape(B, 1)
